# baseline (speedup 1.0000x reference)
import functools

import jax
import jax.numpy as jnp
from jax import lax
from jax.experimental import pallas as pl
from jax.experimental.pallas import tpu as pltpu
from jax.experimental.pallas import tpu_sc as plsc

_MAX_POSITION = 2048
_DEPTH = 64
_LENGTH = 1024

_NC = 2
_NS = 16
_NW = _NC * _NS
_ROWS_PER_W = _LENGTH // _NW
_QS = 8
_BLK = 256
_WIN = 1280
_DG = 16
_FWD_W = 1288
_SUB = 8
_LAN = 128


def _sc_body(tab_hbm, out_hbm, fwd_v, rev_v, wsem):
    cid = lax.axis_index("c")
    sid = lax.axis_index("s")
    wid = sid * _NC + cid
    q = wid % _QS
    blk = wid // _QS
    row0 = blk * _BLK + q

    fwd_lo = pl.multiple_of(1016 + blk * _BLK, 8)
    rev_top = 1279 + q

    lanes = lax.iota(jnp.int32, 16)
    for g in range(_DEPTH // _DG):
        pltpu.sync_copy(
            tab_hbm.at[pl.ds(g * _DG, _DG), pl.ds(fwd_lo, _FWD_W)],
            fwd_v,
        )
        for dl in range(_DG):
            d = g * _DG + dl
            d_idx = jnp.full((16,), dl, jnp.int32)

            def _rev_chunk(ci, carry, d_idx=d_idx, d=d):
                for u in range(4):
                    c = 4 * ci + u
                    src = (rev_top - 16 * c) - lanes
                    rev_v[d // _SUB, d % _SUB,
                          pl.ds(pl.multiple_of(16 * c, 16), 16)] = (
                        plsc.load_gather(fwd_v, [d_idx, src])
                    )
                return carry

            lax.fori_loop(0, _WIN // 64, _rev_chunk, 0)

    def _rows4(t4, carry):
        fired = []
        for tt in range(4):
            t = 4 * t4 + tt
            s = pl.multiple_of((_ROWS_PER_W - 1 - t) * _QS, 8)
            fired += [
                pltpu.async_copy(
                    rev_v.at[:, :, pl.ds(s + jt * _LAN, _LAN)],
                    out_hbm.at[row0 + t * _QS, :, jt],
                    wsem,
                )
                for jt in range(_DEPTH // _SUB)
            ]
        for cp in fired:
            cp.wait()
        return carry

    lax.fori_loop(0, _ROWS_PER_W // 4, _rows4, 0)


@jax.jit
def _rel_pos_sc(table_t):
    mesh = plsc.VectorSubcoreMesh(core_axis_name="c", subcore_axis_name="s")
    return pl.kernel(
        _sc_body,
        out_type=jax.ShapeDtypeStruct(
            (_LENGTH, _SUB, _LENGTH // _LAN, _SUB, _LAN), jnp.float32
        ),
        mesh=mesh,
        scratch_types=[
            pltpu.VMEM((_DG, _FWD_W), jnp.float32),
            pltpu.VMEM((_SUB, _SUB, _WIN), jnp.float32),
            pltpu.SemaphoreType.DMA,
        ],
        compiler_params=pltpu.CompilerParams(
            use_tc_tiling_on_sc=False, needs_layout_passes=False
        ),
    )(table_t)


def kernel(inputs, rel_embeddings):
    del inputs
    table_t = jnp.transpose(rel_embeddings)
    out5 = _rel_pos_sc(table_t)
    out_t = jnp.transpose(out5, (0, 2, 4, 1, 3))
    return jnp.reshape(out_t, (_LENGTH, _LENGTH, _DEPTH))


# speedup vs baseline: 1.0429x; 1.0429x over previous
import functools

import jax
import jax.numpy as jnp
from jax import lax
from jax.experimental import pallas as pl
from jax.experimental.pallas import tpu as pltpu
from jax.experimental.pallas import tpu_sc as plsc

_MAX_POSITION = 2048
_DEPTH = 64
_LENGTH = 1024

_NC = 2
_NS = 16
_NW = _NC * _NS
_ROWS_PER_W = _LENGTH // _NW
_QS = 8
_BLK = 256
_WIN = 1280
_DG = 16
_FWD_W = 1288
_SUB = 8
_LAN = 128


def _sc_body(tab_hbm, out_hbm, fwd_v, rev_v, wsem):
    cid = lax.axis_index("c")
    sid = lax.axis_index("s")
    wid = sid * _NC + cid
    q = wid % _QS
    blk = wid // _QS
    row0 = blk * _BLK + q

    fwd_lo = pl.multiple_of(1016 + blk * _BLK, 8)
    rev_top = 1279 + q

    lanes = lax.iota(jnp.int32, 16)
    for g in range(_DEPTH // _DG):
        pltpu.sync_copy(
            tab_hbm.at[pl.ds(g * _DG, _DG), pl.ds(fwd_lo, _FWD_W)],
            fwd_v,
        )
        for dl in range(_DG):
            d = g * _DG + dl
            d_idx = jnp.full((16,), dl, jnp.int32)

            def _rev_chunk(c, carry, d_idx=d_idx, d=d):
                src = (rev_top - 16 * c) - lanes
                rev_v[d // _SUB, d % _SUB,
                      pl.ds(pl.multiple_of(16 * c, 16), 16)] = (
                    plsc.load_gather(fwd_v, [d_idx, src])
                )
                return carry

            lax.fori_loop(0, _WIN // 16, _rev_chunk, 0)

    def _row(t, carry):
        s = pl.multiple_of((_ROWS_PER_W - 1 - t) * _QS, 8)
        fired = [
            pltpu.async_copy(
                rev_v.at[:, :, pl.ds(s + jt * _LAN, _LAN)],
                out_hbm.at[row0 + t * _QS, :, jt],
                wsem,
            )
            for jt in range(_DEPTH // _SUB)
        ]
        for cp in fired:
            cp.wait()
        return carry

    lax.fori_loop(0, _ROWS_PER_W, _row, 0)


@jax.jit
def _rel_pos_sc(table_t):
    mesh = plsc.VectorSubcoreMesh(core_axis_name="c", subcore_axis_name="s")
    return pl.kernel(
        _sc_body,
        out_type=jax.ShapeDtypeStruct(
            (_LENGTH, _SUB, _LENGTH // _LAN, _SUB, _LAN), jnp.float32
        ),
        mesh=mesh,
        scratch_types=[
            pltpu.VMEM((_DG, _FWD_W), jnp.float32),
            pltpu.VMEM((_SUB, _SUB, _WIN), jnp.float32),
            pltpu.SemaphoreType.DMA,
        ],
        compiler_params=pltpu.CompilerParams(
            use_tc_tiling_on_sc=False, needs_layout_passes=False
        ),
    )(table_t)


def kernel(inputs, rel_embeddings):
    del inputs
    table_t = jnp.transpose(rel_embeddings)
    out5 = _rel_pos_sc(table_t)
    out_t = jnp.transpose(out5, (0, 2, 4, 1, 3))
    return jnp.reshape(out_t, (_LENGTH, _LENGTH, _DEPTH))


# speedup vs baseline: 1.0451x; 1.0021x over previous
import jax
import jax.numpy as jnp
from jax import lax
from jax.experimental import pallas as pl
from jax.experimental.pallas import tpu as pltpu
from jax.experimental.pallas import tpu_sc as plsc

_MAX_POSITION = 2048
_DEPTH = 64
_LENGTH = 1024

_NC = 2
_NS = 16
_NW = _NC * _NS
_ROWS_PER_W = _LENGTH // _NW
_QS = 8
_BLK = 256
_WIN = 1280
_DG = 16
_FWD_W = 1288
_SUB = 8
_LAN = 128


def _sc_body(tab_hbm, out_hbm, fwd_v, rev_v, wsem):
    cid = lax.axis_index("c")
    sid = lax.axis_index("s")
    wid = sid * _NC + cid
    q = wid % _QS
    blk = wid // _QS
    row0 = blk * _BLK + q

    fwd_lo = pl.multiple_of(1016 + blk * _BLK, 8)
    rev_top = 1279 + q

    lanes = lax.iota(jnp.int32, 16)
    for g in range(_DEPTH // _DG):
        pltpu.sync_copy(
            tab_hbm.at[pl.ds(g * _DG, _DG), pl.ds(fwd_lo, _FWD_W)],
            fwd_v,
        )
        for dl in range(_DG):
            d = g * _DG + dl
            d_idx = jnp.full((16,), dl, jnp.int32)

            def _rev_chunk(c, carry, d_idx=d_idx, d=d):
                src = (rev_top - 16 * c) - lanes
                rev_v[d // _SUB, d % _SUB,
                      pl.ds(pl.multiple_of(16 * c, 16), 16)] = (
                    plsc.load_gather(fwd_v, [d_idx, src])
                )
                return carry

            lax.fori_loop(0, _WIN // 16, _rev_chunk, 0)

    def _row(t, carry):
        s = pl.multiple_of((_ROWS_PER_W - 1 - t) * _QS, 8)
        fired = [
            pltpu.async_copy(
                rev_v.at[:, :, pl.ds(s + jt * _LAN, _LAN)],
                out_hbm.at[row0 + t * _QS, :, jt],
                wsem,
            )
            for jt in range(_DEPTH // _SUB)
        ]
        for cp in fired:
            cp.wait()
        return carry

    lax.fori_loop(0, _ROWS_PER_W, _row, 0)


@jax.jit
def _rel_pos_sc(table_t):
    mesh = plsc.VectorSubcoreMesh(core_axis_name="c", subcore_axis_name="s")
    return pl.kernel(
        _sc_body,
        out_type=jax.ShapeDtypeStruct(
            (_LENGTH, _SUB, _LENGTH // _LAN, _SUB, _LAN), jnp.float32
        ),
        mesh=mesh,
        scratch_types=[
            pltpu.VMEM((_DG, _FWD_W), jnp.float32),
            pltpu.VMEM((_SUB, _SUB, _WIN), jnp.float32),
            pltpu.SemaphoreType.DMA,
        ],
        compiler_params=pltpu.CompilerParams(
            use_tc_tiling_on_sc=False, needs_layout_passes=False
        ),
    )(table_t)


def kernel(inputs, rel_embeddings):
    del inputs
    table_t = jnp.transpose(rel_embeddings)
    out5 = _rel_pos_sc(table_t)
    out_t = jnp.transpose(out5, (0, 2, 4, 1, 3))
    return jnp.reshape(out_t, (_LENGTH, _LENGTH, _DEPTH))
